# baseline (speedup 1.0000x reference)
import functools

import jax
import jax.numpy as jnp
from jax import lax
from jax.experimental import pallas as pl
from jax.experimental.pallas import tpu as pltpu
from jax.experimental.pallas import tpu_sc as plsc

_NC = 2
_NS = 16
_NW = _NC * _NS
_L = 16

_BATCH = 16384
_FEAT = 64
_CLASSES = 100000
_FCHUNK = 4096
_NFC = _BATCH // _FCHUNK
_DPW = _FEAT // _NW
_UNROLL = 16
_NACC = 8


def _sc_partials(features_t, labels, centers_t):
    mesh = plsc.VectorSubcoreMesh(
        core_axis_name="c", subcore_axis_name="s",
        num_cores=_NC, num_subcores=_NS,
    )

    @functools.partial(
        pl.kernel,
        out_type=jax.ShapeDtypeStruct((_NW, _L), jnp.float32),
        mesh=mesh,
        scratch_types=[
            pltpu.VMEM((_CLASSES,), jnp.float32),
            pltpu.VMEM((_BATCH,), jnp.int32),
            pltpu.VMEM((2, _FCHUNK), jnp.float32),
            pltpu.VMEM((_L,), jnp.float32),
            pltpu.SemaphoreType.DMA,
            pltpu.SemaphoreType.DMA,
        ],
        compiler_params=pltpu.CompilerParams(
            use_tc_tiling_on_sc=True, needs_layout_passes=False,
            disable_bounds_checks=True, disable_semaphore_checks=True,
            skip_device_barrier=True),
    )
    def k(feat_hbm, lab_hbm, cent_hbm, out_hbm,
          tbl_v, lab_v, fch_v, acc_v, sem_t, sem_f):
        wid = lax.axis_index("s") * _NC + lax.axis_index("c")

        lab_copy = pltpu.async_copy(lab_hbm, lab_v, sem_f)
        lab_copy.wait()

        def feature_dim(r, acc):
            d = wid + r * _NW
            tb = pltpu.async_copy(cent_hbm.at[d], tbl_v, sem_t)
            f0 = pltpu.async_copy(
                feat_hbm.at[d, pl.ds(0, _FCHUNK)], fch_v.at[0], sem_f)
            tb.wait()

            def chunk_body(j, a):
                slot = lax.rem(j, 2)
                nxt = lax.rem(j + 1, 2)
                a = lax.cond(
                    j + 1 < _NFC,
                    lambda x: (pltpu.async_copy(
                        feat_hbm.at[d, pl.ds((j + 1) * _FCHUNK, _FCHUNK)],
                        fch_v.at[nxt], sem_f), x)[1],
                    lambda x: x, a)
                pltpu.make_async_copy(
                    feat_hbm.at[d, pl.ds(0, _FCHUNK)],
                    fch_v.at[slot], sem_f).wait()

                def vec_body(i, a2):
                    a2 = list(a2)
                    for u in range(_UNROLL):
                        off = (i * _UNROLL + u) * _L
                        idx = lab_v[pl.ds(j * _FCHUNK + off, _L)]
                        cv = plsc.load_gather(tbl_v, [idx])
                        fv = fch_v[slot, pl.ds(off, _L)]
                        dlt = fv - cv
                        a2[u % _NACC] = a2[u % _NACC] + dlt * dlt
                    return tuple(a2)

                return lax.fori_loop(
                    0, _FCHUNK // (_L * _UNROLL), vec_body, a)

            return lax.fori_loop(0, _NFC, chunk_body, acc)

        zero = jnp.zeros((_L,), jnp.float32)
        accs = lax.fori_loop(0, _DPW, feature_dim, (zero,) * _NACC)
        total = accs[0]
        for t in accs[1:]:
            total = total + t
        acc_v[...] = total
        pltpu.sync_copy(acc_v, out_hbm.at[wid])

    return k(features_t, labels, centers_t)


def _reduce_body(p_ref, o_ref):
    o_ref[0, 0] = jnp.sum(p_ref[...]) * (0.5 / _BATCH)


def _final_reduce(partials):
    out = pl.pallas_call(
        _reduce_body,
        out_shape=jax.ShapeDtypeStruct((1, 1), jnp.float32),
        out_specs=pl.BlockSpec(memory_space=pltpu.SMEM),
    )(partials)
    return out[0, 0]


def kernel(features, labels, centers):
    labels = labels.astype(jnp.int32)
    partials = _sc_partials(features.T, labels, centers.T)
    return _final_reduce(partials)


# speedup vs baseline: 1.0398x; 1.0398x over previous
import functools

import jax
import jax.numpy as jnp
from jax import lax
from jax.experimental import pallas as pl
from jax.experimental.pallas import tpu as pltpu
from jax.experimental.pallas import tpu_sc as plsc

_NC = 2
_NS = 16
_NW = _NC * _NS
_L = 16

_BATCH = 16384
_FEAT = 64
_CLASSES = 100000
_FCHUNK = 4096
_NFC = _BATCH // _FCHUNK
_DPW = _FEAT // _NW
_UNROLL = 16
_NACC = 8


def _sc_partials(features_t, labels, centers_t):
    mesh = plsc.VectorSubcoreMesh(
        core_axis_name="c", subcore_axis_name="s",
        num_cores=_NC, num_subcores=_NS,
    )

    @functools.partial(
        pl.kernel,
        out_type=jax.ShapeDtypeStruct((_NW, _L), jnp.float32),
        mesh=mesh,
        scratch_types=[
            pltpu.VMEM((_CLASSES,), jnp.float32),
            pltpu.VMEM((_BATCH,), jnp.int32),
            pltpu.VMEM((2, _FCHUNK), jnp.float32),
            pltpu.VMEM((_L,), jnp.float32),
            pltpu.SemaphoreType.DMA,
            pltpu.SemaphoreType.DMA,
        ],
        compiler_params=pltpu.CompilerParams(
            use_tc_tiling_on_sc=True, needs_layout_passes=False),
    )
    def k(feat_hbm, lab_hbm, cent_hbm, out_hbm,
          tbl_v, lab_v, fch_v, acc_v, sem_t, sem_f):
        wid = lax.axis_index("s") * _NC + lax.axis_index("c")

        tb0 = pltpu.async_copy(cent_hbm.at[wid], tbl_v, sem_t)
        lab_copy = pltpu.async_copy(lab_hbm, lab_v, sem_f)
        lab_copy.wait()

        def feature_pass(r, acc):
            d = wid + r * _NW
            if r > 0:
                tb = pltpu.async_copy(cent_hbm.at[d], tbl_v, sem_t)
            else:
                tb = tb0
            f0 = pltpu.async_copy(
                feat_hbm.at[d, pl.ds(0, _FCHUNK)], fch_v.at[0], sem_f)
            tb.wait()

            def chunk_body(j, a):
                slot = lax.rem(j, 2)
                nxt = lax.rem(j + 1, 2)
                a = lax.cond(
                    j + 1 < _NFC,
                    lambda x: (pltpu.async_copy(
                        feat_hbm.at[d, pl.ds((j + 1) * _FCHUNK, _FCHUNK)],
                        fch_v.at[nxt], sem_f), x)[1],
                    lambda x: x, a)
                pltpu.make_async_copy(
                    feat_hbm.at[d, pl.ds(0, _FCHUNK)],
                    fch_v.at[slot], sem_f).wait()

                def vec_body(i, a2):
                    a2 = list(a2)
                    for u in range(_UNROLL):
                        off = (i * _UNROLL + u) * _L
                        idx = lab_v[pl.ds(j * _FCHUNK + off, _L)]
                        cv = plsc.load_gather(tbl_v, [idx])
                        fv = fch_v[slot, pl.ds(off, _L)]
                        dlt = fv - cv
                        a2[u % _NACC] = a2[u % _NACC] + dlt * dlt
                    return tuple(a2)

                return lax.fori_loop(
                    0, _FCHUNK // (_L * _UNROLL), vec_body, a)

            return lax.fori_loop(0, _NFC, chunk_body, acc)

        zero = jnp.zeros((_L,), jnp.float32)
        accs = (zero,) * _NACC
        for r in range(_DPW):
            accs = feature_pass(r, accs)
        total = accs[0]
        for t in accs[1:]:
            total = total + t
        acc_v[...] = total
        pltpu.sync_copy(acc_v, out_hbm.at[wid])

    return k(features_t, labels, centers_t)


def _reduce_body(p_ref, o_ref):
    o_ref[0, 0] = jnp.sum(p_ref[...]) * (0.5 / _BATCH)


def _final_reduce(partials):
    out = pl.pallas_call(
        _reduce_body,
        out_shape=jax.ShapeDtypeStruct((1, 1), jnp.float32),
        out_specs=pl.BlockSpec(memory_space=pltpu.SMEM),
    )(partials)
    return out[0, 0]


def kernel(features, labels, centers):
    labels = labels.astype(jnp.int32)
    partials = _sc_partials(features.T, labels, centers.T)
    return _final_reduce(partials)
